# baseline (speedup 1.0000x reference)
import functools

import jax
import jax.numpy as jnp
from jax.experimental import pallas as pl
from jax.experimental.pallas import tpu as pltpu

TK = 32
NEG_INF = float('-inf')
BIG_I = 2**30


def _body(n_valid, n_blocks, x_ref, w_ref, out_ref, rv_ref, ri_ref):
    qt = x_ref.shape[0]
    bk = w_ref.shape[0]
    j = pl.program_id(1)

    @pl.when(j == 0)
    def _init():
        rv_ref[...] = jnp.full((qt, TK), NEG_INF, jnp.float32)
        ri_ref[...] = jnp.zeros((qt, TK), jnp.int32)

    s = jax.lax.dot_general(
        x_ref[...], w_ref[...], (((1,), (1,)), ((), ())),
        preferred_element_type=jnp.float32)
    col = j * bk + jax.lax.broadcasted_iota(jnp.int32, (qt, bk), 1)
    s = jnp.where(col < n_valid, s, NEG_INF)

    lane = jax.lax.broadcasted_iota(jnp.int32, (qt, TK), 1)

    m0 = jnp.max(s, axis=1, keepdims=True)
    am0 = jnp.min(jnp.where(s == m0, col, BIG_I), axis=1, keepdims=True)

    def cond(c):
        rv, ri, m, am = c
        return jnp.any(m > rv[:, TK - 1:TK])

    def body(c):
        rv, ri, m, am = c
        active = m > rv[:, TK - 1:TK]
        pos = jnp.sum((rv >= m).astype(jnp.int32), axis=1, keepdims=True)
        rv_sh = jnp.concatenate([rv[:, :1], rv[:, :TK - 1]], axis=1)
        ri_sh = jnp.concatenate([ri[:, :1], ri[:, :TK - 1]], axis=1)
        rv_new = jnp.where(lane < pos, rv, jnp.where(lane == pos, m, rv_sh))
        ri_new = jnp.where(lane < pos, ri, jnp.where(lane == pos, am, ri_sh))
        rv = jnp.where(active, rv_new, rv)
        ri = jnp.where(active, ri_new, ri)
        nxt = (s < m) | ((s == m) & (col > am))
        s_eff = jnp.where(nxt, s, NEG_INF)
        m2 = jnp.max(s_eff, axis=1, keepdims=True)
        am2 = jnp.min(jnp.where((s_eff == m2) & nxt, col, BIG_I),
                      axis=1, keepdims=True)
        m = jnp.where(active, m2, NEG_INF)
        return rv, ri, m, am2

    rv, ri, _, _ = jax.lax.while_loop(
        cond, body, (rv_ref[...], ri_ref[...], m0, am0))
    rv_ref[...] = rv
    ri_ref[...] = ri

    @pl.when(j == n_blocks - 1)
    def _out():
        out_ref[...] = ri_ref[...]


def _topk_call(x, w, qt, bk, interpret=False):
    b, d = x.shape
    n = w.shape[0]
    n_blocks = pl.cdiv(n, bk)
    npad = n_blocks * bk
    if npad != n:
        w = jnp.pad(w, ((0, npad - n), (0, 0)))
    grid = (b // qt, n_blocks)
    return pl.pallas_call(
        functools.partial(_body, n, n_blocks),
        grid=grid,
        in_specs=[
            pl.BlockSpec((qt, d), lambda q, j: (q, 0)),
            pl.BlockSpec((bk, d), lambda q, j: (j, 0)),
        ],
        out_specs=pl.BlockSpec((qt, TK), lambda q, j: (q, 0)),
        out_shape=jax.ShapeDtypeStruct((b, TK), jnp.int32),
        scratch_shapes=[
            pltpu.VMEM((qt, TK), jnp.float32),
            pltpu.VMEM((qt, TK), jnp.int32),
        ],
        interpret=interpret,
    )(x, w)


@jax.jit
def kernel(x, W):
    i32 = _topk_call(x, W, qt=128, bk=512)
    return i32.astype(jnp.int64)


# speedup vs baseline: 1.4828x; 1.4828x over previous
import functools

import jax
import jax.numpy as jnp
from jax.experimental import pallas as pl
from jax.experimental.pallas import tpu as pltpu

TK = 32
NEG_INF = float('-inf')
BIG_I = 2**30


def _body(n_valid, n_blocks, x_ref, w_ref, out_ref, rv_ref, ri_ref):
    qt = x_ref.shape[0]
    bk = w_ref.shape[0]
    j = pl.program_id(1)

    @pl.when(j == 0)
    def _init():
        rv_ref[...] = jnp.full((qt, TK), NEG_INF, jnp.float32)
        ri_ref[...] = jnp.zeros((qt, TK), jnp.int32)

    s = jax.lax.dot_general(
        x_ref[...], w_ref[...], (((1,), (1,)), ((), ())),
        preferred_element_type=jnp.float32)
    col = j * bk + jax.lax.broadcasted_iota(jnp.int32, (qt, bk), 1)
    s = jnp.where(col < n_valid, s, NEG_INF)

    lane = jax.lax.broadcasted_iota(jnp.int32, (qt, TK), 1)

    m0 = jnp.max(s, axis=1, keepdims=True)
    am0 = jnp.min(jnp.where(s == m0, col, BIG_I), axis=1, keepdims=True)

    def cond(c):
        rv, ri, m, am = c
        return jnp.any(m > rv[:, TK - 1:TK])

    def body(c):
        rv, ri, m, am = c
        active = m > rv[:, TK - 1:TK]
        pos = jnp.sum((rv >= m).astype(jnp.int32), axis=1, keepdims=True)
        rv_sh = jnp.concatenate([rv[:, :1], rv[:, :TK - 1]], axis=1)
        ri_sh = jnp.concatenate([ri[:, :1], ri[:, :TK - 1]], axis=1)
        rv_new = jnp.where(lane < pos, rv, jnp.where(lane == pos, m, rv_sh))
        ri_new = jnp.where(lane < pos, ri, jnp.where(lane == pos, am, ri_sh))
        rv = jnp.where(active, rv_new, rv)
        ri = jnp.where(active, ri_new, ri)
        nxt = (s < m) | ((s == m) & (col > am))
        s_eff = jnp.where(nxt, s, NEG_INF)
        m2 = jnp.max(s_eff, axis=1, keepdims=True)
        am2 = jnp.min(jnp.where((s_eff == m2) & nxt, col, BIG_I),
                      axis=1, keepdims=True)
        m = jnp.where(active, m2, NEG_INF)
        return rv, ri, m, am2

    rv, ri, _, _ = jax.lax.while_loop(
        cond, body, (rv_ref[...], ri_ref[...], m0, am0))
    rv_ref[...] = rv
    ri_ref[...] = ri

    @pl.when(j == n_blocks - 1)
    def _out():
        out_ref[...] = ri_ref[...]


def _topk_call(x, w, qt, bk, interpret=False):
    b, d = x.shape
    n = w.shape[0]
    n_blocks = pl.cdiv(n, bk)
    npad = n_blocks * bk
    if npad != n:
        w = jnp.pad(w, ((0, npad - n), (0, 0)))
    grid = (b // qt, n_blocks)
    return pl.pallas_call(
        functools.partial(_body, n, n_blocks),
        grid=grid,
        in_specs=[
            pl.BlockSpec((qt, d), lambda q, j: (q, 0)),
            pl.BlockSpec((bk, d), lambda q, j: (j, 0)),
        ],
        out_specs=pl.BlockSpec((qt, TK), lambda q, j: (q, 0)),
        out_shape=jax.ShapeDtypeStruct((b, TK), jnp.int32),
        scratch_shapes=[
            pltpu.VMEM((qt, TK), jnp.float32),
            pltpu.VMEM((qt, TK), jnp.int32),
        ],
        interpret=interpret,
    )(x, w)


@jax.jit
def kernel(x, W):
    i32 = _topk_call(x, W, qt=128, bk=4096)
    return i32.astype(jnp.int64)


# speedup vs baseline: 2.2400x; 1.5107x over previous
import functools

import jax
import jax.numpy as jnp
from jax.experimental import pallas as pl
from jax.experimental.pallas import tpu as pltpu

TK = 32
NEG_INF = float('-inf')
BIG_I = 2**30


def _body(n_valid, nb, x_ref, w_ref, out_ref, l_ref, t_ref, rv_ref, ri_ref):
    qt = x_ref.shape[0]
    bk = w_ref.shape[0]
    j = pl.program_id(1)
    phase1 = j < nb
    b = jnp.where(phase1, j, j - nb)

    s = jax.lax.dot_general(
        x_ref[...], w_ref[...], (((1,), (1,)), ((), ())),
        preferred_element_type=jnp.float32)
    col = b * bk + jax.lax.broadcasted_iota(jnp.int32, (qt, bk), 1)
    s = jnp.where(col < n_valid, s, NEG_INF)

    @pl.when(j == 0)
    def _init_l():
        l_ref[...] = jnp.full((qt, 128), NEG_INF, jnp.float32)

    @pl.when(phase1)
    def _p1():
        l = l_ref[...]
        for c in range(bk // 128):
            l = jnp.maximum(l, s[:, c * 128:(c + 1) * 128])
        l_ref[...] = l

    @pl.when(j == nb - 1)
    def _thresh():
        v = l_ref[...]
        m = jnp.max(v, axis=1, keepdims=True)
        for _ in range(TK - 1):
            v = jnp.where(v == m, NEG_INF, v)
            m = jnp.max(v, axis=1, keepdims=True)
        t_ref[...] = m

    @pl.when(j == nb)
    def _init_r():
        rv_ref[...] = jnp.full((qt, TK), NEG_INF, jnp.float32)
        ri_ref[...] = jnp.zeros((qt, TK), jnp.int32)

    @pl.when(jnp.logical_not(phase1))
    def _p2():
        t = t_ref[...]
        cand = s >= t
        sc = jnp.where(cand, s, NEG_INF)
        cnt = jnp.sum(cand.astype(jnp.int32), axis=1)
        trip = jnp.max(cnt)

        @pl.when(trip > 0)
        def _extract():
            lane = jax.lax.broadcasted_iota(jnp.int32, (qt, TK), 1)
            m0 = jnp.max(sc, axis=1, keepdims=True)
            am0 = jnp.min(jnp.where(sc == m0, col, BIG_I), axis=1,
                          keepdims=True)

            def fbody(_, c):
                rv, ri, m, am = c
                active = m > rv[:, TK - 1:TK]
                pos = jnp.sum((rv >= m).astype(jnp.int32), axis=1,
                              keepdims=True)
                rv_sh = jnp.concatenate([rv[:, :1], rv[:, :TK - 1]], axis=1)
                ri_sh = jnp.concatenate([ri[:, :1], ri[:, :TK - 1]], axis=1)
                rv_new = jnp.where(lane < pos, rv,
                                   jnp.where(lane == pos, m, rv_sh))
                ri_new = jnp.where(lane < pos, ri,
                                   jnp.where(lane == pos, am, ri_sh))
                rv = jnp.where(active, rv_new, rv)
                ri = jnp.where(active, ri_new, ri)
                nxt = (sc < m) | ((sc == m) & (col > am))
                s_eff = jnp.where(nxt, sc, NEG_INF)
                m2 = jnp.max(s_eff, axis=1, keepdims=True)
                am2 = jnp.min(jnp.where((s_eff == m2) & nxt, col, BIG_I),
                              axis=1, keepdims=True)
                return rv, ri, m2, am2

            rv, ri, _, _ = jax.lax.fori_loop(
                0, trip, fbody, (rv_ref[...], ri_ref[...], m0, am0))
            rv_ref[...] = rv
            ri_ref[...] = ri

    @pl.when(j == 2 * nb - 1)
    def _out():
        out_ref[...] = ri_ref[...]


def _topk_call(x, w, qt, bk, interpret=False):
    b, d = x.shape
    n = w.shape[0]
    nb = pl.cdiv(n, bk)
    npad = nb * bk
    if npad != n:
        w = jnp.pad(w, ((0, npad - n), (0, 0)))
    grid = (b // qt, 2 * nb)
    return pl.pallas_call(
        functools.partial(_body, n, nb),
        grid=grid,
        in_specs=[
            pl.BlockSpec((qt, d), lambda q, j: (q, 0)),
            pl.BlockSpec((bk, d),
                         lambda q, j: (jnp.where(j < nb, j, j - nb), 0)),
        ],
        out_specs=pl.BlockSpec((qt, TK), lambda q, j: (q, 0)),
        out_shape=jax.ShapeDtypeStruct((b, TK), jnp.int32),
        scratch_shapes=[
            pltpu.VMEM((qt, 128), jnp.float32),
            pltpu.VMEM((qt, 1), jnp.float32),
            pltpu.VMEM((qt, TK), jnp.float32),
            pltpu.VMEM((qt, TK), jnp.int32),
        ],
        interpret=interpret,
    )(x, w)


@jax.jit
def kernel(x, W):
    i32 = _topk_call(x, W, qt=128, bk=2048)
    return i32.astype(jnp.int64)
